# baseline (speedup 1.0000x reference)
import functools
import jax
import jax.numpy as jnp
from jax import lax
from jax.experimental import pallas as pl
from jax.experimental.pallas import tpu as pltpu
from jax.experimental.pallas import tpu_sc as plsc

BATCH = 16384
RANK = 16

NC = 2
NS = 16
NW = NC * NS
B_PER_W = BATCH // NW
CHUNK = 128
NCHUNK = B_PER_W // CHUNK
NBLK = BATCH // CHUNK


def _gather_body(users_hbm, items_hbm, UT_hbm, VT_hbm, euT_hbm, evT_hbm,
                 uidx, vidx, gu, gv, sem):
    wid = lax.axis_index("s") * NC + lax.axis_index("c")
    cbase = wid * NCHUNK
    pltpu.sync_copy(users_hbm.at[pl.ds(cbase, NCHUNK), :], uidx)
    pltpu.sync_copy(items_hbm.at[pl.ds(cbase, NCHUNK), :], vidx)
    copies = []
    for f in range(RANK):
        for c in range(NCHUNK):
            copies.append(pltpu.async_copy(
                UT_hbm.at[f].at[uidx.at[c]], gu.at[f, c], sem))
            copies.append(pltpu.async_copy(
                VT_hbm.at[f].at[vidx.at[c]], gv.at[f, c], sem))
    for cp in copies:
        cp.wait()
    pltpu.sync_copy(gu, euT_hbm.at[:, pl.ds(cbase, NCHUNK), :])
    pltpu.sync_copy(gv, evT_hbm.at[:, pl.ds(cbase, NCHUNK), :])


@jax.jit
def _sc_gather(users, items, U, V):
    mesh = plsc.VectorSubcoreMesh(core_axis_name="c", subcore_axis_name="s",
                                  num_cores=NC, num_subcores=NS)
    users2 = users.reshape(NBLK, CHUNK)
    items2 = items.reshape(NBLK, CHUNK)
    UT = U.T
    VT = V.T
    out_ty = (
        jax.ShapeDtypeStruct((RANK, NBLK, CHUNK), jnp.float32),
        jax.ShapeDtypeStruct((RANK, NBLK, CHUNK), jnp.float32),
    )
    scratch = [
        pltpu.VMEM((NCHUNK, CHUNK), jnp.int32),
        pltpu.VMEM((NCHUNK, CHUNK), jnp.int32),
        pltpu.VMEM((RANK, NCHUNK, CHUNK), jnp.float32),
        pltpu.VMEM((RANK, NCHUNK, CHUNK), jnp.float32),
        pltpu.SemaphoreType.DMA,
    ]
    euT3, evT3 = pl.kernel(_gather_body, out_type=out_ty, mesh=mesh,
                           scratch_types=scratch,
                           compiler_params=pltpu.CompilerParams(
                               use_tc_tiling_on_sc=False))(
                                   users2, items2, UT, VT)
    return euT3.reshape(RANK, BATCH), evT3.reshape(RANK, BATCH)


MLP_BLOCK = 2048


def _mlp_body(euT_ref, evT_ref, w1a_ref, w1b_ref, b1_ref, w2_ref, b2_ref,
              w3_ref, b3_ref, out_ref):
    h = jnp.dot(w1a_ref[...], euT_ref[...], preferred_element_type=jnp.float32)
    h = h + jnp.dot(w1b_ref[...], evT_ref[...],
                    preferred_element_type=jnp.float32)
    h = jnp.maximum(h + b1_ref[...], 0.0)
    h = jnp.dot(w2_ref[...], h, preferred_element_type=jnp.float32) + b2_ref[...]
    h = jnp.maximum(h, 0.0)
    out_ref[...] = (jnp.dot(w3_ref[...], h, preferred_element_type=jnp.float32)
                    + b3_ref[...])


@jax.jit
def _tc_mlp(euT, evT, W1, b1, W2, b2, W3, b3):
    w1a = W1[:, :RANK]
    w1b = W1[:, RANK:]
    b1c = b1.reshape(-1, 1)
    b2c = b2.reshape(-1, 1)
    b3c = b3.reshape(-1, 1)
    grid = BATCH // MLP_BLOCK
    full = lambda s: pl.BlockSpec(s, lambda i: (0,) * len(s))
    out = pl.pallas_call(
        _mlp_body,
        grid=(grid,),
        in_specs=[
            pl.BlockSpec((RANK, MLP_BLOCK), lambda i: (0, i)),
            pl.BlockSpec((RANK, MLP_BLOCK), lambda i: (0, i)),
            full(w1a.shape), full(w1b.shape), full(b1c.shape),
            full(W2.shape), full(b2c.shape), full(W3.shape), full(b3c.shape),
        ],
        out_specs=pl.BlockSpec((1, MLP_BLOCK), lambda i: (0, i)),
        out_shape=jax.ShapeDtypeStruct((1, BATCH), jnp.float32),
    )(euT, evT, w1a, w1b, b1c, W2, b2c, W3, b3c)
    return out[0]


def kernel(users, items, U, V, W1, b1, W2, b2, W3, b3):
    users = users.astype(jnp.int32)
    items = items.astype(jnp.int32)
    euT, evT = _sc_gather(users, items, U, V)
    return _tc_mlp(euT, evT, W1, b1, W2, b2, W3, b3)


# speedup vs baseline: 2.5180x; 2.5180x over previous
import functools
import jax
import jax.numpy as jnp
from jax import lax
from jax.experimental import pallas as pl
from jax.experimental.pallas import tpu as pltpu
from jax.experimental.pallas import tpu_sc as plsc

BATCH = 16384
RANK = 16
HALF = 500000

NC = 2
NS = 16
NW = NC * NS
B_PER_W = BATCH // NW
CHUNK = 128
NCHUNK = B_PER_W // CHUNK


def _gather_body(users_hbm, items0_hbm, items1_hbm, U_hbm, V0_hbm, V1_hbm,
                 eu_hbm, g0_hbm, g1_hbm,
                 uidx, v0idx, v1idx, urows, v0rows, v1rows, sem):
    wid = lax.axis_index("s") * NC + lax.axis_index("c")
    pltpu.sync_copy(users_hbm.at[wid], uidx)
    pltpu.sync_copy(items0_hbm.at[wid], v0idx)
    pltpu.sync_copy(items1_hbm.at[wid], v1idx)
    copies = []
    for j in range(NCHUNK):
        copies.append(pltpu.async_copy(U_hbm.at[uidx.at[j]], urows.at[j], sem))
        copies.append(pltpu.async_copy(V0_hbm.at[v0idx.at[j]], v0rows.at[j], sem))
        copies.append(pltpu.async_copy(V1_hbm.at[v1idx.at[j]], v1rows.at[j], sem))
    for c in copies:
        c.wait()
    pltpu.sync_copy(urows, eu_hbm.at[wid])
    pltpu.sync_copy(v0rows, g0_hbm.at[wid])
    pltpu.sync_copy(v1rows, g1_hbm.at[wid])


@jax.jit
def _sc_gather(users, items0, items1, U, V0, V1):
    mesh = plsc.VectorSubcoreMesh(core_axis_name="c", subcore_axis_name="s",
                                  num_cores=NC, num_subcores=NS)
    users3 = users.reshape(NW, NCHUNK, CHUNK)
    items03 = items0.reshape(NW, NCHUNK, CHUNK)
    items13 = items1.reshape(NW, NCHUNK, CHUNK)
    out_ty = tuple(
        jax.ShapeDtypeStruct((NW, NCHUNK, CHUNK, RANK), jnp.float32)
        for _ in range(3)
    )
    scratch = [
        pltpu.VMEM((NCHUNK, CHUNK), jnp.int32),
        pltpu.VMEM((NCHUNK, CHUNK), jnp.int32),
        pltpu.VMEM((NCHUNK, CHUNK), jnp.int32),
        pltpu.VMEM((NCHUNK, CHUNK, RANK), jnp.float32),
        pltpu.VMEM((NCHUNK, CHUNK, RANK), jnp.float32),
        pltpu.VMEM((NCHUNK, CHUNK, RANK), jnp.float32),
        pltpu.SemaphoreType.DMA,
    ]
    eu, g0, g1 = pl.kernel(_gather_body, out_type=out_ty, mesh=mesh,
                           scratch_types=scratch,
                           compiler_params=pltpu.CompilerParams(
                               use_tc_tiling_on_sc=False))(
                                   users3, items03, items13, U, V0, V1)
    return (eu.reshape(BATCH, RANK), g0.reshape(BATCH, RANK),
            g1.reshape(BATCH, RANK))


MLP_BLOCK = 2048


def _mlp_body(eu_ref, g0_ref, g1_ref, it_ref, w1aT_ref, w1bT_ref, b1_ref,
              w2T_ref, b2_ref, w3T_ref, b3_ref, out_ref):
    ev = jnp.where(it_ref[...] < HALF, g0_ref[...], g1_ref[...])
    h = jnp.dot(eu_ref[...], w1aT_ref[...], preferred_element_type=jnp.float32)
    h = h + jnp.dot(ev, w1bT_ref[...], preferred_element_type=jnp.float32)
    h = jnp.maximum(h + b1_ref[...], 0.0)
    h = jnp.dot(h, w2T_ref[...], preferred_element_type=jnp.float32) + b2_ref[...]
    h = jnp.maximum(h, 0.0)
    out_ref[...] = (jnp.dot(h, w3T_ref[...], preferred_element_type=jnp.float32)
                    + b3_ref[...])


@jax.jit
def _tc_mlp(eu, g0, g1, items, W1, b1, W2, b2, W3, b3):
    w1aT = W1[:, :RANK].T
    w1bT = W1[:, RANK:].T
    w2T = W2.T
    w3T = W3.T
    b1r = b1.reshape(1, -1)
    b2r = b2.reshape(1, -1)
    b3r = b3.reshape(1, -1)
    it2 = items.reshape(BATCH, 1)
    grid = BATCH // MLP_BLOCK
    full = lambda s: pl.BlockSpec(s, lambda i: (0,) * len(s))
    row_blk = pl.BlockSpec((MLP_BLOCK, RANK), lambda i: (i, 0))
    out = pl.pallas_call(
        _mlp_body,
        grid=(grid,),
        in_specs=[
            row_blk, row_blk, row_blk,
            pl.BlockSpec((MLP_BLOCK, 1), lambda i: (i, 0)),
            full(w1aT.shape), full(w1bT.shape), full(b1r.shape),
            full(w2T.shape), full(b2r.shape), full(w3T.shape), full(b3r.shape),
        ],
        out_specs=pl.BlockSpec((MLP_BLOCK, 1), lambda i: (i, 0)),
        out_shape=jax.ShapeDtypeStruct((BATCH, 1), jnp.float32),
    )(eu, g0, g1, it2, w1aT, w1bT, b1r, w2T, b2r, w3T, b3r)
    return out[:, 0]


def kernel(users, items, U, V, W1, b1, W2, b2, W3, b3):
    users = users.astype(jnp.int32)
    items = items.astype(jnp.int32)
    items0 = jnp.minimum(items, HALF - 1)
    items1 = jnp.clip(items - HALF, 0, V.shape[0] - HALF - 1)
    V0 = V[:HALF]
    V1 = V[HALF:]
    eu, g0, g1 = _sc_gather(users, items0, items1, U, V0, V1)
    return _tc_mlp(eu, g0, g1, items, W1, b1, W2, b2, W3, b3)


# speedup vs baseline: 2.6749x; 1.0623x over previous
import functools
import jax
import jax.numpy as jnp
from jax import lax
from jax.experimental import pallas as pl
from jax.experimental.pallas import tpu as pltpu
from jax.experimental.pallas import tpu_sc as plsc

BATCH = 16384
RANK = 16

NC = 2
NS = 16
NW = NC * NS
B_PER_W = BATCH // NW
CHUNK = 128
NCHUNK = B_PER_W // CHUNK


def _gather_body(users_hbm, items_hbm, U_hbm, V_hbm, eu_hbm, ev_hbm,
                 uidx, vidx, urows, vrows, sem):
    wid = lax.axis_index("s") * NC + lax.axis_index("c")
    pltpu.sync_copy(users_hbm.at[wid], uidx)
    pltpu.sync_copy(items_hbm.at[wid], vidx)
    copies = []
    for j in range(NCHUNK):
        copies.append(pltpu.async_copy(U_hbm.at[uidx.at[j]], urows.at[j], sem))
        copies.append(pltpu.async_copy(V_hbm.at[vidx.at[j]], vrows.at[j], sem))
    for c in copies:
        c.wait()
    pltpu.sync_copy(urows, eu_hbm.at[wid])
    pltpu.sync_copy(vrows, ev_hbm.at[wid])


@jax.jit
def _sc_gather(users, items, U, V):
    mesh = plsc.VectorSubcoreMesh(core_axis_name="c", subcore_axis_name="s",
                                  num_cores=NC, num_subcores=NS)
    users3 = users.reshape(NW, NCHUNK, CHUNK)
    items3 = items.reshape(NW, NCHUNK, CHUNK)
    out_ty = (
        jax.ShapeDtypeStruct((NW, NCHUNK, CHUNK, RANK), jnp.float32),
        jax.ShapeDtypeStruct((NW, NCHUNK, CHUNK, RANK), jnp.float32),
    )
    scratch = [
        pltpu.VMEM((NCHUNK, CHUNK), jnp.int32),
        pltpu.VMEM((NCHUNK, CHUNK), jnp.int32),
        pltpu.VMEM((NCHUNK, CHUNK, RANK), jnp.float32),
        pltpu.VMEM((NCHUNK, CHUNK, RANK), jnp.float32),
        pltpu.SemaphoreType.DMA,
    ]
    eu, ev = pl.kernel(_gather_body, out_type=out_ty, mesh=mesh,
                       scratch_types=scratch,
                       compiler_params=pltpu.CompilerParams(
                           use_tc_tiling_on_sc=False))(users3, items3, U, V)
    return eu.reshape(BATCH, RANK), ev.reshape(BATCH, RANK)


MLP_BLOCK = 2048


def _mlp_body(eu_ref, ev_ref, w1aT_ref, w1bT_ref, b1_ref, w2T_ref, b2_ref,
              w3T_ref, b3_ref, out_ref):
    h = jnp.dot(eu_ref[...], w1aT_ref[...], preferred_element_type=jnp.float32)
    h = h + jnp.dot(ev_ref[...], w1bT_ref[...], preferred_element_type=jnp.float32)
    h = jnp.maximum(h + b1_ref[...], 0.0)
    h = jnp.dot(h, w2T_ref[...], preferred_element_type=jnp.float32) + b2_ref[...]
    h = jnp.maximum(h, 0.0)
    out_ref[...] = (jnp.dot(h, w3T_ref[...], preferred_element_type=jnp.float32)
                    + b3_ref[...])


@jax.jit
def _tc_mlp(eu, ev, W1, b1, W2, b2, W3, b3):
    w1aT = W1[:, :RANK].T
    w1bT = W1[:, RANK:].T
    w2T = W2.T
    w3T = W3.T
    b1r = b1.reshape(1, -1)
    b2r = b2.reshape(1, -1)
    b3r = b3.reshape(1, -1)
    grid = BATCH // MLP_BLOCK
    full = lambda s: pl.BlockSpec(s, lambda i: (0,) * len(s))
    out = pl.pallas_call(
        _mlp_body,
        grid=(grid,),
        in_specs=[
            pl.BlockSpec((MLP_BLOCK, RANK), lambda i: (i, 0)),
            pl.BlockSpec((MLP_BLOCK, RANK), lambda i: (i, 0)),
            full(w1aT.shape), full(w1bT.shape), full(b1r.shape),
            full(w2T.shape), full(b2r.shape), full(w3T.shape), full(b3r.shape),
        ],
        out_specs=pl.BlockSpec((MLP_BLOCK, 1), lambda i: (i, 0)),
        out_shape=jax.ShapeDtypeStruct((BATCH, 1), jnp.float32),
    )(eu, ev, w1aT, w1bT, b1r, w2T, b2r, w3T, b3r)
    return out[:, 0]


def kernel(users, items, U, V, W1, b1, W2, b2, W3, b3):
    users = users.astype(jnp.int32)
    items = items.astype(jnp.int32)
    eu, ev = _sc_gather(users, items, U, V)
    return _tc_mlp(eu, ev, W1, b1, W2, b2, W3, b3)
